# baseline (speedup 1.0000x reference)
import functools
import jax
import jax.numpy as jnp
from jax import lax
from jax.experimental import pallas as pl
from jax.experimental.pallas import tpu as pltpu
from jax.experimental.pallas import tpu_sc as plsc

H = 256
BN = 2000
GK = 40
NTILES = 32
SG = 8
SEGU = NTILES * SG * 16
RING = 8

_mesh = functools.partial(
    plsc.VectorSubcoreMesh, core_axis_name="c", subcore_axis_name="s")


def _ln_silu(t, g, b):
    mu = jnp.mean(t, axis=-1, keepdims=True)
    var = jnp.mean((t - mu) ** 2, axis=-1, keepdims=True)
    t = (t - mu) * jax.lax.rsqrt(var + 1e-5) * g + b
    return t * jax.nn.sigmoid(t)


def _atom_embed_body(x_ref, w_ref, b_ref, g_ref, bb_ref, o_ref):
    t = jnp.dot(x_ref[...], w_ref[...], preferred_element_type=jnp.float32)
    o_ref[...] = _ln_silu(t + b_ref[...], g_ref[...], bb_ref[...])


def _tc_atom_embed(x, p):
    n, din = x.shape
    return pl.pallas_call(
        _atom_embed_body,
        grid=(n // BN,),
        in_specs=[
            pl.BlockSpec((BN, din), lambda i: (i, 0)),
            pl.BlockSpec((din, H), lambda i: (0, 0)),
            pl.BlockSpec((1, H), lambda i: (0, 0)),
            pl.BlockSpec((1, H), lambda i: (0, 0)),
            pl.BlockSpec((1, H), lambda i: (0, 0)),
        ],
        out_specs=pl.BlockSpec((BN, H), lambda i: (i, 0)),
        out_shape=jax.ShapeDtypeStruct((n, H), jnp.float32),
    )(x, p["W"], p["b"][None], p["ln_g"][None], p["ln_b"][None])


def _rbf_mlp2_body(nbins, vmin, vmax, clip,
                   d_ref, w1, b1, g1, bb1, w2, b2, g2, bb2, o_ref):
    d = d_ref[...]
    if clip:
        d = jnp.clip(d, vmin, vmax)
    centers = vmin + (vmax - vmin) / (nbins - 1) * lax.broadcasted_iota(
        jnp.int32, (1, nbins), 1).astype(jnp.float32)
    gamma = 1.0 / ((vmax - vmin) / (nbins - 1)) ** 2
    r = jnp.exp(-gamma * (d - centers) ** 2)
    t = jnp.dot(r, w1[...], preferred_element_type=jnp.float32)
    t = _ln_silu(t + b1[...], g1[...], bb1[...])
    t = jnp.dot(t, w2[...], preferred_element_type=jnp.float32)
    o_ref[...] = _ln_silu(t + b2[...], g2[...], bb2[...])


def _tc_rbf_mlp2(d, p1, p2, nbins, vmin, vmax, clip):
    n = d.shape[0]
    dmid = p1["W"].shape[1]
    return pl.pallas_call(
        functools.partial(_rbf_mlp2_body, nbins, vmin, vmax, clip),
        grid=(n // BN,),
        in_specs=[
            pl.BlockSpec((BN, 1), lambda i: (i, 0)),
            pl.BlockSpec((nbins, dmid), lambda i: (0, 0)),
            pl.BlockSpec((1, dmid), lambda i: (0, 0)),
            pl.BlockSpec((1, dmid), lambda i: (0, 0)),
            pl.BlockSpec((1, dmid), lambda i: (0, 0)),
            pl.BlockSpec((dmid, H), lambda i: (0, 0)),
            pl.BlockSpec((1, H), lambda i: (0, 0)),
            pl.BlockSpec((1, H), lambda i: (0, 0)),
            pl.BlockSpec((1, H), lambda i: (0, 0)),
        ],
        out_specs=pl.BlockSpec((BN, H), lambda i: (i, 0)),
        out_shape=jax.ShapeDtypeStruct((n, H), jnp.float32),
    )(d[:, None], p1["W"], p1["b"][None], p1["ln_g"][None], p1["ln_b"][None],
      p2["W"], p2["b"][None], p2["ln_g"][None], p2["ln_b"][None])


def _linear4_body(h_ref, w_ref, b_ref, a_ref, bo_ref, d_ref, e_ref):
    t = jnp.dot(h_ref[...], w_ref[...], preferred_element_type=jnp.float32)
    t = t + b_ref[...]
    a_ref[...] = t[:, 0 * H:1 * H]
    bo_ref[...] = t[:, 1 * H:2 * H]
    d_ref[...] = t[:, 2 * H:3 * H]
    e_ref[...] = t[:, 3 * H:4 * H]


def _tc_linear4(h, p):
    n = h.shape[0]
    w = jnp.concatenate([p["src_gate"]["W"], p["dst_gate"]["W"],
                         p["dst_update"]["W"], p["src_update"]["W"]], axis=1)
    b = jnp.concatenate([p["src_gate"]["b"], p["dst_gate"]["b"],
                         p["dst_update"]["b"], p["src_update"]["b"]])[None]
    outs = [jax.ShapeDtypeStruct((n, H), jnp.float32)] * 4
    return pl.pallas_call(
        _linear4_body,
        grid=(n // BN,),
        in_specs=[
            pl.BlockSpec((BN, H), lambda i: (i, 0)),
            pl.BlockSpec((H, 4 * H), lambda i: (0, 0)),
            pl.BlockSpec((1, 4 * H), lambda i: (0, 0)),
        ],
        out_specs=[pl.BlockSpec((BN, H), lambda i: (i, 0))] * 4,
        out_shape=outs,
    )(h, w, b)


def _edge_combine_body(nb, e_ref, pa_ref, pb_ref, q_ref, w_ref, b_ref,
                       g_ref, bb_ref, sig_ref, num_ref, eo_ref):
    i = pl.program_id(0)

    @pl.when(i < nb)
    def _():
        m = jnp.dot(e_ref[...], w_ref[...],
                    preferred_element_type=jnp.float32)
        m = m + b_ref[...] + pa_ref[...] + pb_ref[...]
        sig = jax.nn.sigmoid(m)
        sig_ref[...] = sig
        num_ref[...] = q_ref[...] * sig
        eo_ref[...] = e_ref[...] + _ln_silu(m, g_ref[...], bb_ref[...])

    @pl.when(i == nb)
    def _():
        sig_ref[...] = jnp.zeros_like(sig_ref)
        num_ref[...] = jnp.zeros_like(num_ref)


def _tc_edge_combine(e, pa, pb, q, p):
    n = e.shape[0]
    nb = n // BN
    clamp = lambda i: (jnp.minimum(i, nb - 1), 0)
    outs = [jax.ShapeDtypeStruct((n + BN, H), jnp.float32)] * 2 + [
        jax.ShapeDtypeStruct((n, H), jnp.float32)]
    return pl.pallas_call(
        functools.partial(_edge_combine_body, nb),
        grid=(nb + 1,),
        in_specs=[pl.BlockSpec((BN, H), clamp)] * 4 + [
            pl.BlockSpec((H, H), lambda i: (0, 0)),
            pl.BlockSpec((1, H), lambda i: (0, 0)),
            pl.BlockSpec((1, H), lambda i: (0, 0)),
            pl.BlockSpec((1, H), lambda i: (0, 0)),
        ],
        out_specs=[pl.BlockSpec((BN, H), lambda i: (i, 0))] * 2 + [
            pl.BlockSpec((BN, H), clamp)],
        out_shape=outs,
    )(e, pa, pb, q, p["edge_gate"]["W"], p["edge_gate"]["b"][None],
      p["ln_e_g"][None], p["ln_e_b"][None])


def _node_update_body(eu_ref, ss_ref, sn_ref, h_ref, g_ref, bb_ref, o_ref):
    xn = eu_ref[...] + sn_ref[...] / (ss_ref[...] + 1e-6)
    o_ref[...] = h_ref[...] + _ln_silu(xn, g_ref[...], bb_ref[...])


def _tc_node_update(eu, ss, sn, h, p, bn):
    n = h.shape[0]
    return pl.pallas_call(
        _node_update_body,
        grid=(n // bn,),
        in_specs=[pl.BlockSpec((bn, H), lambda i: (i, 0))] * 4 + [
            pl.BlockSpec((1, H), lambda i: (0, 0)),
            pl.BlockSpec((1, H), lambda i: (0, 0)),
        ],
        out_specs=pl.BlockSpec((bn, H), lambda i: (i, 0)),
        out_shape=jax.ShapeDtypeStruct((n, H), jnp.float32),
    )(eu, ss, sn, h, p["ln_n_g"][None], p["ln_n_b"][None])


def _readout_body(h_ref, w_ref, b_ref, o_ref, acc_ref):
    i = pl.program_id(0)

    @pl.when(i == 0)
    def _():
        acc_ref[...] = jnp.zeros_like(acc_ref)

    acc_ref[...] += jnp.sum(h_ref[...], axis=0, keepdims=True)

    @pl.when(i == pl.num_programs(0) - 1)
    def _():
        hg = acc_ref[...] / h_ref.shape[0] / pl.num_programs(0)
        o_ref[...] = jnp.dot(hg, w_ref[...],
                             preferred_element_type=jnp.float32) + b_ref[...]


def _tc_readout(h, p):
    n = h.shape[0]
    out = pl.pallas_call(
        _readout_body,
        grid=(n // BN,),
        in_specs=[
            pl.BlockSpec((BN, H), lambda i: (i, 0)),
            pl.BlockSpec((H, 1), lambda i: (0, 0)),
            pl.BlockSpec((1, 1), lambda i: (0, 0)),
        ],
        out_specs=pl.BlockSpec((1, 1), lambda i: (0, 0)),
        out_shape=jax.ShapeDtypeStruct((1, 1), jnp.float32),
        scratch_shapes=[pltpu.VMEM((1, H), jnp.float32)],
    )(h, p["W"], p["b"][None])
    return out[0, 0]


def _gather3_body(ne, a_hbm, b_hbm, d_hbm, u_hbm, v_hbm,
                  pa_hbm, pb_hbm, q_hbm,
                  idxu, idxv, bufa, bufb, bufd, sem):
    wid = lax.axis_index("s") * 2 + lax.axis_index("c")
    per = ne // NTILES
    base = wid * per

    def step(i, carry):
        p = pl.multiple_of(base + i * GK, 8)
        pltpu.sync_copy(u_hbm.at[pl.ds(p, GK)], idxu)
        pltpu.sync_copy(v_hbm.at[pl.ds(p, GK)], idxv)
        ca = pltpu.make_async_copy(a_hbm.at[idxu], bufa, sem)
        cb = pltpu.make_async_copy(b_hbm.at[idxv], bufb, sem)
        cd = pltpu.make_async_copy(d_hbm.at[idxu], bufd, sem)
        ca.start()
        cb.start()
        cd.start()
        ca.wait()
        cb.wait()
        cd.wait()
        pltpu.sync_copy(bufa, pa_hbm.at[pl.ds(p, GK)])
        pltpu.sync_copy(bufb, pb_hbm.at[pl.ds(p, GK)])
        pltpu.sync_copy(bufd, q_hbm.at[pl.ds(p, GK)])
        return carry

    lax.fori_loop(0, per // GK, step, 0)


def _sc_gather3(a, b, d, u, v):
    ne = u.shape[0]
    outs = [jax.ShapeDtypeStruct((ne, H), jnp.float32)] * 3
    f = pl.kernel(
        functools.partial(_gather3_body, ne),
        out_type=outs,
        mesh=_mesh(),
        scratch_types=[
            pltpu.VMEM((GK,), jnp.int32),
            pltpu.VMEM((GK,), jnp.int32),
            pltpu.VMEM((GK, H), jnp.float32),
            pltpu.VMEM((GK, H), jnp.float32),
            pltpu.VMEM((GK, H), jnp.float32),
            pltpu.SemaphoreType.DMA,
        ],
    )
    return f(a, b, d, u, v)


def _segsum2_body(nseg_pad, ne,
                  sig_hbm, num_hbm, r0_hbm, len_hbm, gmax_hbm, zero_hbm,
                  ss_hbm, sn_hbm,
                  r0v, lenv, gmaxv, idxb, sem_s, sem_n, acc_s, acc_n):
    wid = lax.axis_index("s") * 2 + lax.axis_index("c")
    ngroups = nseg_pad // (SG * 16)
    per_w = ngroups // NTILES
    lane = lax.broadcasted_iota(jnp.int32, (16,), 0)

    def group(g_i, carry):
        g = wid * per_w + g_i
        seg0 = g * SG * 16
        pltpu.sync_copy(r0_hbm.at[pl.ds(seg0, SG * 16)], r0v)
        pltpu.sync_copy(len_hbm.at[pl.ds(seg0, SG * 16)], lenv)
        pltpu.sync_copy(gmax_hbm.at[pl.ds(g * 16, 16)], gmaxv)
        pltpu.sync_copy(zero_hbm, acc_s)
        pltpu.sync_copy(zero_hbm, acc_n)
        kmax = gmaxv[...][0]

        def waitall(_):
            for j in range(SG):
                pltpu.make_async_copy(
                    sig_hbm.at[pl.ds(0, 16)],
                    acc_s.at[pl.ds(j * 16, 16)], sem_s).wait()
                pltpu.make_async_copy(
                    num_hbm.at[pl.ds(0, 16)],
                    acc_n.at[pl.ds(j * 16, 16)], sem_n).wait()

        def rnd(k, carry2):
            @pl.when(k >= RING)
            def _():
                waitall(None)
            slot = (k % RING) * (SG * 16)
            for j in range(SG):
                r0j = r0v[pl.ds(j * 16, 16)]
                lnj = lenv[pl.ds(j * 16, 16)]
                dummy = ne + ((wid * SG + j) * 16) % BN + lane
                idxb[pl.ds(slot + j * 16, 16)] = jnp.where(
                    k < lnj, r0j + k, dummy)
            for j in range(SG):
                pltpu.async_copy(
                    sig_hbm.at[idxb.at[pl.ds(slot + j * 16, 16)]],
                    acc_s.at[pl.ds(j * 16, 16)], sem_s, add=True)
                pltpu.async_copy(
                    num_hbm.at[idxb.at[pl.ds(slot + j * 16, 16)]],
                    acc_n.at[pl.ds(j * 16, 16)], sem_n, add=True)
            return carry2

        lax.fori_loop(0, kmax, rnd, 0)
        lax.fori_loop(0, jnp.minimum(kmax, RING),
                      lambda t, c: (waitall(None), c)[1], 0)

        pltpu.sync_copy(acc_s, ss_hbm.at[pl.ds(seg0, SG * 16)])
        pltpu.sync_copy(acc_n, sn_hbm.at[pl.ds(seg0, SG * 16)])
        return carry

    lax.fori_loop(0, per_w, group, 0)


def _sc_segsum2(sig, num, r0, lens, gmax16, zeros):
    nseg_pad = r0.shape[0]
    ne = sig.shape[0] - BN
    outs = [jax.ShapeDtypeStruct((nseg_pad, H), jnp.float32)] * 2
    f = pl.kernel(
        functools.partial(_segsum2_body, nseg_pad, ne),
        out_type=outs,
        mesh=_mesh(),
        scratch_types=[
            pltpu.VMEM((SG * 16,), jnp.int32),
            pltpu.VMEM((SG * 16,), jnp.int32),
            pltpu.VMEM((16,), jnp.int32),
            pltpu.VMEM((RING * SG * 16,), jnp.int32),
            pltpu.SemaphoreType.DMA,
            pltpu.SemaphoreType.DMA,
            pltpu.VMEM((SG * 16, H), jnp.float32),
            pltpu.VMEM((SG * 16, H), jnp.float32),
        ],
    )
    return f(sig, num, r0, lens, gmax16, zeros)


def _prep_seg(sv, nseg):
    nseg_pad = -(-max(nseg, 1) // SEGU) * SEGU
    r0 = jnp.searchsorted(
        sv, jnp.arange(nseg_pad + 1, dtype=jnp.int32)).astype(jnp.int32)
    lens = r0[1:] - r0[:-1]
    gmax = jnp.max(lens.reshape(-1, SG * 16), axis=1)
    gmax16 = jnp.repeat(gmax, 16)
    return r0[:nseg_pad], lens, gmax16


def _eggc(p, u, v, meta, h, e, zeros):
    r0, lens, gmax16 = meta
    a, b, d, eu = _tc_linear4(h, p)
    pa, pb, q = _sc_gather3(a, b, d, u, v)
    sig, num, e_out = _tc_edge_combine(e, pa, pb, q, p)
    ss, sn = _sc_segsum2(sig, num, r0, lens, gmax16, zeros)
    h_out = _tc_node_update(eu, ss, sn, h, p, BN)
    return h_out, e_out


def kernel(x, edge_index, bondlength, lg_edge_index, angle, params):
    n = x.shape[0]
    ne = bondlength.shape[0]
    u = edge_index[0].astype(jnp.int32)
    v = edge_index[1].astype(jnp.int32)
    lu = lg_edge_index[0].astype(jnp.int32)
    lv = lg_edge_index[1].astype(jnp.int32)

    perm_e = jnp.argsort(v).astype(jnp.int32)
    u_s = u[perm_e]
    v_s = v[perm_e]
    inv_e = jnp.argsort(perm_e).astype(jnp.int32)
    perm_lg = jnp.argsort(inv_e[lv]).astype(jnp.int32)
    lu_s = inv_e[lu][perm_lg]
    lv_s = inv_e[lv][perm_lg]

    meta_n = _prep_seg(v_s, n)
    meta_e = _prep_seg(lv_s, ne)
    zeros = jnp.zeros((SG * 16, H), jnp.float32)

    h = _tc_atom_embed(x, params["atom_emb"])
    y = _tc_rbf_mlp2(bondlength[perm_e], params["edge_emb1"],
                     params["edge_emb2"], 80, 0.0, 8.0, False)
    z = _tc_rbf_mlp2(angle[perm_lg], params["angle_emb1"],
                     params["angle_emb2"], 40, -1.0, 1.0, True)

    for layer in params["alignn"]:
        h, m = _eggc(layer["node"], u_s, v_s, meta_n, h, y, zeros)
        y, z = _eggc(layer["edge"], lu_s, lv_s, meta_e, m, z, zeros)
    for p in params["gcn"]:
        h, y = _eggc(p, u_s, v_s, meta_n, h, y, zeros)

    return _tc_readout(h, params["fc"])
